# baseline (speedup 1.0000x reference)
import numpy as np
import jax
import jax.numpy as jnp
from jax.experimental import pallas as pl
from jax.experimental.pallas import tpu as pltpu

LENS_L = 7
_R = LENS_L // 2
H = 384
W = 384
WP = 512
RB = 128
RBP = RB + 2 * _R
NRB = H // RB
EPS = 1e-8

_cls = {}
for _u in range(-_R, _R + 1):
    for _v in range(-_R, _R + 1):
        _dsq = _u * _u + _v * _v
        if np.float32(np.sqrt(_dsq)) <= _R + 1e-6:
            _cls.setdefault(_dsq, []).append((_u, _v))
CLASSES = [(np.float32(np.sqrt(d)), len(t), t) for d, t in sorted(_cls.items())]

VPAT = {}
for _ci, (_dist, _cnt, _taps) in enumerate(CLASSES):
    for (_u, _v) in _taps:
        if _v >= 0:
            VPAT.setdefault(_v, []).append((_u, _ci))
for _v in VPAT:
    VPAT[_v].sort()
assert sum(len(p) * (2 if v > 0 else 1) for v, p in VPAT.items()) == 29


def _render_kernel(lens_ref, x_ref, f_ref, out_ref, win_ref, tq_ref, sh_ref):
    b = pl.program_id(0)
    rb = pl.program_id(1)
    le = lens_ref[b]
    row0 = rb * RB
    nch = x_ref.shape[1]
    n_layer = nch // 5

    def copy_rows(src_start, voff, nload, dst_start, nrows):
        for ch in range(nch):
            src = (x_ref[0, ch, pl.ds(src_start, nload), :]
                   - f_ref[0, ch, pl.ds(src_start, nload), :])
            if voff or nload != nrows:
                src = jax.lax.slice(src, (voff, 0), (voff + nrows, W))
            left = jnp.broadcast_to(src[:, 0:1], (nrows, _R))
            right = jnp.broadcast_to(src[:, W - 1:W], (nrows, WP - W - _R))
            win_ref[ch, pl.ds(dst_start, nrows), :] = jnp.concatenate(
                [left, src, right], axis=1)

    def fill_rows(dst_start, nfill, src_row):
        for ch in range(nch):
            rowv = win_ref[ch, src_row:src_row + 1, :]
            win_ref[ch, pl.ds(dst_start, nfill), :] = jnp.broadcast_to(
                rowv, (nfill, WP))

    @pl.when(rb == 0)
    def _():
        copy_rows(0, 0, RBP - _R, _R, RBP - _R)
        fill_rows(0, _R, _R)

    @pl.when(rb == NRB - 1)
    def _():
        copy_rows(row0 - 8, 8 - _R, RBP - _R + (8 - _R), 0, RBP - _R)
        fill_rows(RBP - _R, _R, RBP - _R - 1)

    if NRB > 2:
        @pl.when(jnp.logical_and(rb != 0, rb != NRB - 1))
        def _():
            copy_rows(row0 - 8, 8 - _R, RBP + 8, 0, RBP)

    blur_rgb = None
    trans = None
    for li in range(n_layer):
        rgb = [win_ref[5 * li + c] for c in range(3)]
        a = win_ref[5 * li + 3]
        d = win_ref[5 * li + 4]
        t = jnp.abs(d * le) + 0.5
        norm = jnp.full_like(t, np.float32(CLASSES[0][1]))
        for dist, count, _taps in CLASSES[1:]:
            norm = norm + jnp.where(t >= dist, np.float32(count), 0.0)
        inv = 1.0 / (norm + EPS)

        wa = a * inv
        tq_ref[0] = t
        tq_ref[1] = rgb[0] * wa
        tq_ref[2] = rgb[1] * wa
        tq_ref[3] = rgb[2] * wa
        tq_ref[4] = wa

        for uo in range(LENS_L):
            sh_ref[uo] = tq_ref[:, pl.ds(uo, RB), :]

        S = {}
        for av, pat in VPAT.items():
            s_ch = [None] * 4
            for (u, ci) in pat:
                uo = _R + u
                dist = CLASSES[ci][0]
                tsl = sh_ref[uo, 0]
                for ch in range(4):
                    if ci == 0:
                        term = sh_ref[uo, 1 + ch]
                    else:
                        term = jnp.where(tsl >= dist, sh_ref[uo, 1 + ch], 0.0)
                    s_ch[ch] = term if s_ch[ch] is None else s_ch[ch] + term
            S[av] = s_ch

        acc = [None] * 4
        for v in range(-_R, _R + 1):
            s_ch = S[abs(v)]
            for ch in range(4):
                term = jax.lax.slice(s_ch[ch], (0, _R + v), (RB, _R + v + W))
                acc[ch] = term if acc[ch] is None else acc[ch] + term

        ow = acc[3]
        occu = jnp.clip(ow, 0.0, 1.0)
        scale = occu / (ow + EPS)
        layer_rgb = [acc[c] * scale for c in range(3)]
        if li == 0:
            blur_rgb = layer_rgb
            trans = 1.0 - occu
        else:
            blur_rgb = [blur_rgb[c] + layer_rgb[c] * trans for c in range(3)]
            trans = trans * (1.0 - occu)

    out_ref[0] = jnp.stack(blur_rgb, axis=0)


def kernel(rgbad_layers, lens_effect, focal):
    B, C5, _, _ = rgbad_layers.shape
    le = lens_effect.reshape(B)

    out = pl.pallas_call(
        _render_kernel,
        grid=(B, NRB),
        in_specs=[
            pl.BlockSpec(memory_space=pltpu.SMEM),
            pl.BlockSpec((1, C5, H, W), lambda b, rb: (b, 0, 0, 0)),
            pl.BlockSpec((1, C5, H, W), lambda b, rb: (b, 0, 0, 0)),
        ],
        out_specs=pl.BlockSpec((1, 3, RB, W), lambda b, rb: (b, 0, rb, 0)),
        out_shape=jax.ShapeDtypeStruct((B, 3, H, W), jnp.float32),
        scratch_shapes=[pltpu.VMEM((C5, RBP, WP), jnp.float32),
                        pltpu.VMEM((5, RBP, WP), jnp.float32),
                        pltpu.VMEM((LENS_L, 5, RB, WP), jnp.float32)],
    )(le, rgbad_layers, focal)
    return out


# speedup vs baseline: 1.0164x; 1.0164x over previous
import numpy as np
import jax
import jax.numpy as jnp
from jax.experimental import pallas as pl
from jax.experimental.pallas import tpu as pltpu

LENS_L = 7
_R = LENS_L // 2
H = 384
W = 384
HP = 392
WP = 512
RB = 128
RBP = RB + 2 * _R
NRB = H // RB
EPS = 1e-8

_cls = {}
for _u in range(-_R, _R + 1):
    for _v in range(-_R, _R + 1):
        _dsq = _u * _u + _v * _v
        if np.float32(np.sqrt(_dsq)) <= _R + 1e-6:
            _cls.setdefault(_dsq, []).append((_u, _v))
CLASSES = [(np.float32(np.sqrt(d)), len(t), t) for d, t in sorted(_cls.items())]

VPAT = {}
for _ci, (_dist, _cnt, _taps) in enumerate(CLASSES):
    for (_u, _v) in _taps:
        if _v >= 0:
            VPAT.setdefault(_v, []).append((_u, _ci))
for _v in VPAT:
    VPAT[_v].sort()
assert sum(len(p) * (2 if v > 0 else 1) for v, p in VPAT.items()) == 29


def _render_kernel(lens_ref, x_ref, f_ref, out_ref, win_ref, tq_ref, sh_ref):
    b = pl.program_id(0)
    rb = pl.program_id(1)
    le = lens_ref[b]
    row0 = rb * RB
    nch = x_ref.shape[1]
    n_layer = nch // 5

    @pl.when(rb == 0)
    def _():
        for ch in range(nch):
            src = x_ref[0, ch] - f_ref[0, ch]
            left = jnp.broadcast_to(src[:, 0:1], (H, _R))
            right = jnp.broadcast_to(src[:, W - 1:W], (H, WP - W - _R))
            win_ref[ch, _R:_R + H, :] = jnp.concatenate([left, src, right],
                                                        axis=1)
            top = win_ref[ch, _R:_R + 1, :]
            win_ref[ch, 0:_R, :] = jnp.broadcast_to(top, (_R, WP))
            bot = win_ref[ch, _R + H - 1:_R + H, :]
            win_ref[ch, _R + H:HP, :] = jnp.broadcast_to(bot, (HP - _R - H, WP))

    blur_rgb = None
    trans = None
    for li in range(n_layer):
        rgb = [win_ref[5 * li + c, pl.ds(row0, RBP), :] for c in range(3)]
        a = win_ref[5 * li + 3, pl.ds(row0, RBP), :]
        d = win_ref[5 * li + 4, pl.ds(row0, RBP), :]
        t = jnp.abs(d * le) + 0.5
        norm = jnp.full_like(t, np.float32(CLASSES[0][1]))
        for dist, count, _taps in CLASSES[1:]:
            norm = norm + jnp.where(t >= dist, np.float32(count), 0.0)
        inv = 1.0 / (norm + EPS)

        wa = a * inv
        tq_ref[0] = t
        tq_ref[1] = rgb[0] * wa
        tq_ref[2] = rgb[1] * wa
        tq_ref[3] = rgb[2] * wa
        tq_ref[4] = wa

        for uo in range(LENS_L):
            sh_ref[uo] = tq_ref[:, pl.ds(uo, RB), :]

        S = {}
        for av, pat in VPAT.items():
            s = None
            for (u, ci) in pat:
                uo = _R + u
                qblk = sh_ref[uo, 1:5]
                if ci == 0:
                    term = qblk
                else:
                    m = sh_ref[uo, 0] >= CLASSES[ci][0]
                    term = jnp.where(m[None, :, :], qblk, 0.0)
                s = term if s is None else s + term
            S[av] = s

        acc = None
        for v in range(-_R, _R + 1):
            term = jax.lax.slice(S[abs(v)], (0, 0, _R + v), (4, RB, _R + v + W))
            acc = term if acc is None else acc + term

        ow = acc[3]
        occu = jnp.clip(ow, 0.0, 1.0)
        scale = occu / (ow + EPS)
        layer_rgb = acc[0:3] * scale[None]
        if li == 0:
            blur_rgb = layer_rgb
            trans = 1.0 - occu
        else:
            blur_rgb = blur_rgb + layer_rgb * trans[None]
            trans = trans * (1.0 - occu)

    out_ref[0] = blur_rgb


def kernel(rgbad_layers, lens_effect, focal):
    B, C5, _, _ = rgbad_layers.shape
    le = lens_effect.reshape(B)

    out = pl.pallas_call(
        _render_kernel,
        grid=(B, NRB),
        in_specs=[
            pl.BlockSpec(memory_space=pltpu.SMEM),
            pl.BlockSpec((1, C5, H, W), lambda b, rb: (b, 0, 0, 0)),
            pl.BlockSpec((1, C5, H, W), lambda b, rb: (b, 0, 0, 0)),
        ],
        out_specs=pl.BlockSpec((1, 3, RB, W), lambda b, rb: (b, 0, rb, 0)),
        out_shape=jax.ShapeDtypeStruct((B, 3, H, W), jnp.float32),
        scratch_shapes=[pltpu.VMEM((C5, HP, WP), jnp.float32),
                        pltpu.VMEM((5, RBP, WP), jnp.float32),
                        pltpu.VMEM((LENS_L, 5, RB, WP), jnp.float32)],
    )(le, rgbad_layers, focal)
    return out
